# baseline (speedup 1.0000x reference)
import jax
import jax.numpy as jnp
from jax import lax
from jax.experimental import pallas as pl
from jax.experimental.pallas import tpu as pltpu

_B = 128
_V = 100000
_EPS = 1e-10
_BLK = 8192
_NB = (_V + _BLK - 1) // _BLK
_RH = 64
_RG = 64
_SLAB = 512
_BIG = 2147483647

_KS0 = 0
_KS1 = 42
_KS2 = 0x1BD11BDA ^ 42
_ROT1 = (13, 15, 26, 6)
_ROT2 = (17, 29, 16, 24)


def _rotl(x, r):
    return lax.shift_left(x, jnp.int32(r)) | lax.shift_right_logical(
        x, jnp.int32(32 - r)
    )


def _threefry_bits(n):
    x0 = jnp.zeros_like(n)
    x1 = n + jnp.int32(_KS1)

    def rounds(x0, x1, rots):
        for r in rots:
            x0 = x0 + x1
            x1 = _rotl(x1, r)
            x1 = x1 ^ x0
        return x0, x1

    x0, x1 = rounds(x0, x1, _ROT1)
    x0 = x0 + jnp.int32(_KS1)
    x1 = x1 + jnp.int32(_KS2 + 1)
    x0, x1 = rounds(x0, x1, _ROT2)
    x0 = x0 + jnp.int32(_KS2)
    x1 = x1 + jnp.int32(_KS0 + 2)
    x0, x1 = rounds(x0, x1, _ROT1)
    x0 = x0 + jnp.int32(_KS0)
    x1 = x1 + jnp.int32(_KS1 + 3)
    x0, x1 = rounds(x0, x1, _ROT2)
    x0 = x0 + jnp.int32(_KS1)
    x1 = x1 + jnp.int32(_KS2 + 4)
    x0, x1 = rounds(x0, x1, _ROT1)
    x0 = x0 + jnp.int32(_KS2)
    x1 = x1 + jnp.int32(_KS0 + 5)
    return x0 ^ x1


def _greedy_body(l_ref, max_ref, idx_ref, m_scr, i_scr):
    v = pl.program_id(1)

    @pl.when(v == 0)
    def _():
        m_scr[...] = jnp.full_like(m_scr, -jnp.inf)
        i_scr[...] = jnp.zeros_like(i_scr)

    l = l_ref[...]
    col = lax.broadcasted_iota(jnp.int32, l.shape, 1) + v * _BLK
    lm = jnp.where(col < _V, l, -jnp.inf)
    bm = jnp.max(lm, axis=1, keepdims=True)
    bi = jnp.min(jnp.where(lm == bm, col, _BIG), axis=1, keepdims=True)
    upd = bm > m_scr[...]
    i_scr[...] = jnp.where(upd, bi, i_scr[...])
    m_scr[...] = jnp.where(upd, bm, m_scr[...])

    @pl.when(v == _NB - 1)
    def _():
        max_ref[...] = m_scr[...]
        idx_ref[...] = i_scr[...]


def _race_body(l_ref, t_ref, m_ref, g_ref, out_ref, m_scr, i_scr):
    r = pl.program_id(0)
    v = pl.program_id(1)

    @pl.when(v == 0)
    def _():
        m_scr[...] = jnp.full_like(m_scr, -jnp.inf)
        i_scr[...] = jnp.zeros_like(i_scr)

    for s in range(_RG // 8):
        t = t_ref[s * 8:(s + 1) * 8]
        m = m_ref[s * 8:(s + 1) * 8]
        for c in range(_BLK // _SLAB):
            l = l_ref[s * 8:(s + 1) * 8, c * _SLAB:(c + 1) * _SLAB]
            row = (lax.broadcasted_iota(jnp.int32, l.shape, 0)
                   + r * _RG + s * 8)
            col = (lax.broadcasted_iota(jnp.int32, l.shape, 1)
                   + v * _BLK + c * _SLAB)
            bits = _threefry_bits(row * _V + col)
            f = lax.bitcast_convert_type(
                lax.shift_right_logical(bits, jnp.int32(9))
                | jnp.int32(0x3F800000),
                jnp.float32,
            )
            e = -jnp.log1p(1.0 - f)
            val = jnp.exp(l / t - m) / (e + _EPS)
            val = jnp.where(col < _V, val, -1.0)
            bm = jnp.max(val, axis=1, keepdims=True)
            bi = jnp.min(jnp.where(val == bm, col, _BIG), axis=1,
                         keepdims=True)
            upd = bm > m_scr[s * 8:(s + 1) * 8]
            i_scr[s * 8:(s + 1) * 8] = jnp.where(upd, bi,
                                                 i_scr[s * 8:(s + 1) * 8])
            m_scr[s * 8:(s + 1) * 8] = jnp.where(upd, bm,
                                                 m_scr[s * 8:(s + 1) * 8])

    @pl.when(v == _NB - 1)
    def _():
        out_ref[...] = jnp.where(t_ref[...] == 0.0, g_ref[...], i_scr[...])


def kernel(logits, temperatures):
    logits = logits.astype(jnp.float32)

    row_spec = pl.BlockSpec((_RH, 1), lambda r, v: (r, 0))
    blk_spec = pl.BlockSpec((_RH, _BLK), lambda r, v: (r, v))

    lmax, gidx = pl.pallas_call(
        _greedy_body,
        grid=(_B // _RH, _NB),
        in_specs=[blk_spec],
        out_specs=[row_spec, row_spec],
        out_shape=[
            jax.ShapeDtypeStruct((_B, 1), jnp.float32),
            jax.ShapeDtypeStruct((_B, 1), jnp.int32),
        ],
        scratch_shapes=[
            pltpu.VMEM((_RH, 1), jnp.float32),
            pltpu.VMEM((_RH, 1), jnp.int32),
        ],
        compiler_params=pltpu.CompilerParams(
            dimension_semantics=("parallel", "arbitrary"),
        ),
    )(logits)

    t_col = temperatures[:, None]
    m_col = lmax / t_col

    row_spec2 = pl.BlockSpec((_RG, 1), lambda r, v: (r, 0))
    out = pl.pallas_call(
        _race_body,
        grid=(_B // _RG, _NB),
        in_specs=[
            pl.BlockSpec((_RG, _BLK), lambda r, v: (r, v)),
            row_spec2,
            row_spec2,
            row_spec2,
        ],
        out_specs=row_spec2,
        out_shape=jax.ShapeDtypeStruct((_B, 1), jnp.int32),
        scratch_shapes=[
            pltpu.VMEM((_RG, 1), jnp.float32),
            pltpu.VMEM((_RG, 1), jnp.int32),
        ],
        compiler_params=pltpu.CompilerParams(
            dimension_semantics=("arbitrary", "arbitrary"),
        ),
    )(logits, t_col, m_col, gidx)

    return out[:, 0]


# speedup vs baseline: 1.0463x; 1.0463x over previous
import jax
import jax.numpy as jnp
from jax import lax
from jax.experimental import pallas as pl
from jax.experimental.pallas import tpu as pltpu

_B = 128
_V = 100000
_EPS = 1e-10
_BLK = 8192
_NB = (_V + _BLK - 1) // _BLK
_RH = 64
_RG = 32
_SLAB = 512
_BIG = 2147483647

_KS0 = 0
_KS1 = 42
_KS2 = 0x1BD11BDA ^ 42
_ROT1 = (13, 15, 26, 6)
_ROT2 = (17, 29, 16, 24)


def _rotl(x, r):
    return lax.shift_left(x, jnp.int32(r)) | lax.shift_right_logical(
        x, jnp.int32(32 - r)
    )


def _threefry_bits(n):
    x0 = jnp.zeros_like(n)
    x1 = n + jnp.int32(_KS1)

    def rounds(x0, x1, rots):
        for r in rots:
            x0 = x0 + x1
            x1 = _rotl(x1, r)
            x1 = x1 ^ x0
        return x0, x1

    x0, x1 = rounds(x0, x1, _ROT1)
    x0 = x0 + jnp.int32(_KS1)
    x1 = x1 + jnp.int32(_KS2 + 1)
    x0, x1 = rounds(x0, x1, _ROT2)
    x0 = x0 + jnp.int32(_KS2)
    x1 = x1 + jnp.int32(_KS0 + 2)
    x0, x1 = rounds(x0, x1, _ROT1)
    x0 = x0 + jnp.int32(_KS0)
    x1 = x1 + jnp.int32(_KS1 + 3)
    x0, x1 = rounds(x0, x1, _ROT2)
    x0 = x0 + jnp.int32(_KS1)
    x1 = x1 + jnp.int32(_KS2 + 4)
    x0, x1 = rounds(x0, x1, _ROT1)
    x0 = x0 + jnp.int32(_KS2)
    x1 = x1 + jnp.int32(_KS0 + 5)
    return x0 ^ x1


def _greedy_body(l_ref, max_ref, idx_ref, m_scr, i_scr):
    v = pl.program_id(1)

    @pl.when(v == 0)
    def _():
        m_scr[...] = jnp.full_like(m_scr, -jnp.inf)
        i_scr[...] = jnp.zeros_like(i_scr)

    l = l_ref[...]
    col = lax.broadcasted_iota(jnp.int32, l.shape, 1) + v * _BLK
    lm = jnp.where(col < _V, l, -jnp.inf)
    bm = jnp.max(lm, axis=1, keepdims=True)
    bi = jnp.min(jnp.where(lm == bm, col, _BIG), axis=1, keepdims=True)
    upd = bm > m_scr[...]
    i_scr[...] = jnp.where(upd, bi, i_scr[...])
    m_scr[...] = jnp.where(upd, bm, m_scr[...])

    @pl.when(v == _NB - 1)
    def _():
        max_ref[...] = m_scr[...]
        idx_ref[...] = i_scr[...]


def _race_body(l_ref, t_ref, m_ref, g_ref, out_ref, m_scr, i_scr):
    r = pl.program_id(0)
    v = pl.program_id(1)

    @pl.when(v == 0)
    def _():
        m_scr[...] = jnp.full_like(m_scr, -jnp.inf)
        i_scr[...] = jnp.zeros_like(i_scr)

    for s in range(_RG // 8):
        t = t_ref[s * 8:(s + 1) * 8]
        m = m_ref[s * 8:(s + 1) * 8]
        for c in range(_BLK // _SLAB):
            l = l_ref[s * 8:(s + 1) * 8, c * _SLAB:(c + 1) * _SLAB]
            row = (lax.broadcasted_iota(jnp.int32, l.shape, 0)
                   + r * _RG + s * 8)
            col = (lax.broadcasted_iota(jnp.int32, l.shape, 1)
                   + v * _BLK + c * _SLAB)
            bits = _threefry_bits(row * _V + col)
            f = lax.bitcast_convert_type(
                lax.shift_right_logical(bits, jnp.int32(9))
                | jnp.int32(0x3F800000),
                jnp.float32,
            )
            e = -jnp.log1p(1.0 - f)
            val = jnp.exp(l / t - m) / (e + _EPS)
            val = jnp.where(col < _V, val, -1.0)
            bm = jnp.max(val, axis=1, keepdims=True)
            bi = jnp.min(jnp.where(val == bm, col, _BIG), axis=1,
                         keepdims=True)
            upd = bm > m_scr[s * 8:(s + 1) * 8]
            i_scr[s * 8:(s + 1) * 8] = jnp.where(upd, bi,
                                                 i_scr[s * 8:(s + 1) * 8])
            m_scr[s * 8:(s + 1) * 8] = jnp.where(upd, bm,
                                                 m_scr[s * 8:(s + 1) * 8])

    @pl.when(v == _NB - 1)
    def _():
        out_ref[...] = jnp.where(t_ref[...] == 0.0, g_ref[...], i_scr[...])


def kernel(logits, temperatures):
    logits = logits.astype(jnp.float32)

    row_spec = pl.BlockSpec((_RH, 1), lambda r, v: (r, 0))
    blk_spec = pl.BlockSpec((_RH, _BLK), lambda r, v: (r, v))

    lmax, gidx = pl.pallas_call(
        _greedy_body,
        grid=(_B // _RH, _NB),
        in_specs=[blk_spec],
        out_specs=[row_spec, row_spec],
        out_shape=[
            jax.ShapeDtypeStruct((_B, 1), jnp.float32),
            jax.ShapeDtypeStruct((_B, 1), jnp.int32),
        ],
        scratch_shapes=[
            pltpu.VMEM((_RH, 1), jnp.float32),
            pltpu.VMEM((_RH, 1), jnp.int32),
        ],
        compiler_params=pltpu.CompilerParams(
            dimension_semantics=("parallel", "arbitrary"),
        ),
    )(logits)

    t_col = temperatures[:, None]
    m_col = lmax / t_col

    row_spec2 = pl.BlockSpec((_RG, 1), lambda r, v: (r, 0))
    out = pl.pallas_call(
        _race_body,
        grid=(_B // _RG, _NB),
        in_specs=[
            pl.BlockSpec((_RG, _BLK), lambda r, v: (r, v)),
            row_spec2,
            row_spec2,
            row_spec2,
        ],
        out_specs=row_spec2,
        out_shape=jax.ShapeDtypeStruct((_B, 1), jnp.int32),
        scratch_shapes=[
            pltpu.VMEM((_RG, 1), jnp.float32),
            pltpu.VMEM((_RG, 1), jnp.int32),
        ],
        compiler_params=pltpu.CompilerParams(
            dimension_semantics=("arbitrary", "arbitrary"),
        ),
    )(logits, t_col, m_col, gidx)

    return out[:, 0]
